# Initial kernel scaffold; baseline (speedup 1.0000x reference)
import jax
import jax.numpy as jnp
from jax.experimental import pallas as pl


def kernel(x, last_update, edge_index, t, msg, W_t, b_t, Wq, bq, Wk, bk, Wv, bv, We, Wskip, bskip):
    raise NotImplementedError("write your pallas kernel here")


# speedup vs baseline: 18.6097x; 18.6097x over previous
import jax
import jax.numpy as jnp
import numpy as np
from jax import lax
from jax.experimental import pallas as pl
from jax.experimental.pallas import tpu as pltpu
from jax.experimental.pallas import tpu_sc as plsc

N = 10000
E = 320000
D = 128
MSG = 16
TIME = 16
H = 2
C = 64
HC = H * C

NW = 32
EPW = E // NW
CH = 80
NCH = EPW // CH
NP = 10240
TPS = NP // 16


def _tc1_body(x_ref, wk_ref, wv_ref, wq_ref, ws_ref, bk_ref, bv_ref, bq_ref, bs_ref,
              kv_ref, q_ref, s_ref):
    x = x_ref[...]
    kv_ref[:, :HC] = jnp.dot(x, wk_ref[...], preferred_element_type=jnp.float32) + bk_ref[...]
    kv_ref[:, HC:] = jnp.dot(x, wv_ref[...], preferred_element_type=jnp.float32) + bv_ref[...]
    q_ref[...] = jnp.dot(x, wq_ref[...], preferred_element_type=jnp.float32) + bq_ref[...]
    s_ref[...] = jnp.dot(x, ws_ref[...], preferred_element_type=jnp.float32) + bs_ref[...]


def _tc1(x, Wk, bk, Wv, bv, Wq, bq, Wskip, bskip):
    BLK = 1000
    grid = (N // BLK,)
    w_spec = pl.BlockSpec((D, HC), lambda i: (0, 0))
    b_spec = pl.BlockSpec((HC,), lambda i: (0,))
    kv, q, s = pl.pallas_call(
        _tc1_body,
        grid=grid,
        in_specs=[pl.BlockSpec((BLK, D), lambda i: (i, 0)),
                  w_spec, w_spec, w_spec, w_spec,
                  b_spec, b_spec, b_spec, b_spec],
        out_specs=[pl.BlockSpec((BLK, 2 * HC), lambda i: (i, 0)),
                   pl.BlockSpec((BLK, HC), lambda i: (i, 0)),
                   pl.BlockSpec((BLK, HC), lambda i: (i, 0))],
        out_shape=[jax.ShapeDtypeStruct((N, 2 * HC), jnp.float32),
                   jax.ShapeDtypeStruct((N, HC), jnp.float32),
                   jax.ShapeDtypeStruct((N, HC), jnp.float32)],
    )(x, Wk, Wv, Wq, Wskip, bk, bv, bq, bskip)
    return kv, q, s


def _sc0_body(lu_hbm, src_hbm, out_hbm, idx_v, lu_v, sem):
    c = lax.axis_index("c")
    s = lax.axis_index("s")
    wid = c * 16 + s
    pltpu.sync_copy(src_hbm.at[wid], idx_v)

    def chunk(j, carry):
        pltpu.async_copy(lu_hbm.at[idx_v.at[j]], lu_v.at[j], sem).wait()
        return carry

    lax.fori_loop(0, NCH, chunk, 0)
    pltpu.sync_copy(lu_v, out_hbm.at[wid])


def _sc0(last_update, src3):
    mesh = plsc.VectorSubcoreMesh(core_axis_name="c", subcore_axis_name="s")
    f = pl.kernel(
        _sc0_body,
        mesh=mesh,
        out_type=jax.ShapeDtypeStruct((NW, NCH, CH), jnp.float32),
        scratch_types=[
            pltpu.VMEM((NCH, CH), jnp.int32),
            pltpu.VMEM((NCH, CH), jnp.float32),
            pltpu.SemaphoreType.DMA,
        ],
    )
    return f(last_update, src3)


def _tc2_body(lu_ref, t_ref, msg_ref, wt_ref, bt_ref, we_ref, e_ref):
    rel = lu_ref[...] - t_ref[...]
    enc = jnp.cos(rel[:, None] * wt_ref[0][None, :] + bt_ref[...][None, :])
    e_ref[...] = (jnp.dot(enc, we_ref[:TIME, :], preferred_element_type=jnp.float32)
                  + jnp.dot(msg_ref[...], we_ref[TIME:, :], preferred_element_type=jnp.float32))


def _tc2(lu_src, t, msg, W_t, b_t, We):
    BLK = 512
    grid = (E // BLK,)
    return pl.pallas_call(
        _tc2_body,
        grid=grid,
        in_specs=[pl.BlockSpec((BLK,), lambda i: (i,)),
                  pl.BlockSpec((BLK,), lambda i: (i,)),
                  pl.BlockSpec((BLK, MSG), lambda i: (i, 0)),
                  pl.BlockSpec((1, TIME), lambda i: (0, 0)),
                  pl.BlockSpec((TIME,), lambda i: (0,)),
                  pl.BlockSpec((TIME + MSG, HC), lambda i: (0, 0))],
        out_specs=pl.BlockSpec((BLK, HC), lambda i: (i, 0)),
        out_shape=jax.ShapeDtypeStruct((E, HC), jnp.float32),
    )(lu_src, t, msg, W_t, b_t, We)


DN_ROWS = 256


def _sc1_body(kv_hbm, q_hbm, e_hbm, src_hbm, dst_hbm, parts_hbm, dn_hbm,
              sidx_c, didx_c, kv_v, q_v, e_v, d64_v, zrow_v,
              acc_sh, dn_sh, sem_kv, sem_q, sem_e):
    c = lax.axis_index("c")
    s = lax.axis_index("s")
    wid = c * 16 + s

    def zlane(i, carry):
        for j in range(HC // 16):
            zrow_v[i, pl.ds(j * 16, 16)] = jnp.zeros((16,), jnp.float32)
        return carry

    lax.fori_loop(0, 16, zlane, 0)

    def zacc(r, carry):
        pltpu.sync_copy(zrow_v, acc_sh.at[pl.ds(s * TPS + r * 16, 16)])
        return carry

    lax.fori_loop(0, TPS // 16, zacc, 0)
    pltpu.sync_copy(zrow_v, dn_sh.at[pl.ds(s * 16, 16)])
    plsc.subcore_barrier()

    inv_sqrt_c = np.float32(1.0 / np.sqrt(C))
    lane = lax.iota(jnp.int32, 16)

    def lane_sum(vec):
        for sft in (1, 2, 4, 8):
            vec = vec + vec.at[jnp.bitwise_xor(lane, sft)].get(mode="promise_in_bounds")
        return vec

    def chunk(j, carry):
        ebase = wid * EPW + j * CH
        pltpu.sync_copy(src_hbm.at[pl.ds(ebase, CH)], sidx_c)
        pltpu.sync_copy(dst_hbm.at[pl.ds(ebase, CH)], didx_c)
        cp_kv = pltpu.async_copy(kv_hbm.at[sidx_c], kv_v, sem_kv)
        cp_q = pltpu.async_copy(q_hbm.at[didx_c], q_v, sem_q)
        cp_e = pltpu.async_copy(e_hbm.at[pl.ds(ebase, CH)], e_v, sem_e)
        cp_kv.wait()
        cp_q.wait()
        cp_e.wait()

        for g in range(CH // 16):
            d16 = didx_c[pl.ds(g * 16, 16)]
            d64_v[pl.ds(g * 16, 16)] = lax.shift_right_logical(d16, 6)

        def edge(i, carry2):
            acc0 = jnp.zeros((16,), jnp.float32)
            acc1 = jnp.zeros((16,), jnp.float32)
            for jj in range(4):
                qv = q_v[i, pl.ds(jj * 16, 16)]
                kv = kv_v[i, pl.ds(jj * 16, 16)] + e_v[i, pl.ds(jj * 16, 16)]
                acc0 = acc0 + qv * kv
            for jj in range(4, 8):
                qv = q_v[i, pl.ds(jj * 16, 16)]
                kv = kv_v[i, pl.ds(jj * 16, 16)] + e_v[i, pl.ds(jj * 16, 16)]
                acc1 = acc1 + qv * kv
            a0v = jnp.exp(lane_sum(acc0) * inv_sqrt_c)
            a1v = jnp.exp(lane_sum(acc1) * inv_sqrt_c)
            for jj in range(4):
                e_v[i, pl.ds(jj * 16, 16)] = a0v * (
                    kv_v[i, pl.ds(HC + jj * 16, 16)] + e_v[i, pl.ds(jj * 16, 16)])
            for jj in range(4, 8):
                e_v[i, pl.ds(jj * 16, 16)] = a1v * (
                    kv_v[i, pl.ds(HC + jj * 16, 16)] + e_v[i, pl.ds(jj * 16, 16)])
            dgrp = didx_c[pl.ds((i // 16) * 16, 16)]
            dstv = dgrp.at[jnp.full((16,), i % 16, jnp.int32)].get(
                mode="promise_in_bounds")
            col0 = 2 * jnp.bitwise_and(dstv, 63)
            zero16 = jnp.zeros((16,), jnp.float32)
            for jj in range(8):
                lane16 = lane + (16 * jj)
                q_v[i, pl.ds(jj * 16, 16)] = jnp.where(
                    lane16 == col0, a0v,
                    jnp.where(lane16 == col0 + 1, a1v, zero16))
            return carry2

        lax.fori_loop(0, CH, edge, 0)
        pltpu.sync_copy(e_v, acc_sh.at[didx_c], add=True)
        pltpu.sync_copy(q_v, dn_sh.at[d64_v], add=True)
        return carry

    lax.fori_loop(0, NCH, chunk, 0)
    plsc.subcore_barrier()

    pltpu.sync_copy(acc_sh.at[pl.ds(s * TPS, TPS)],
                    parts_hbm.at[c, pl.ds(s * TPS, TPS)])
    pltpu.sync_copy(dn_sh.at[pl.ds(s * 16, 16)],
                    dn_hbm.at[c, pl.ds(s * 16, 16)])


def _sc1(kv, q, e_full, src3, dst3):
    mesh = plsc.VectorSubcoreMesh(core_axis_name="c", subcore_axis_name="s")
    f = pl.kernel(
        _sc1_body,
        mesh=mesh,
        out_type=[jax.ShapeDtypeStruct((2, NP, HC), jnp.float32),
                  jax.ShapeDtypeStruct((2, DN_ROWS, HC), jnp.float32)],
        scratch_types=[
            pltpu.VMEM((CH,), jnp.int32),
            pltpu.VMEM((CH,), jnp.int32),
            pltpu.VMEM((CH, 2 * HC), jnp.float32),
            pltpu.VMEM((CH, HC), jnp.float32),
            pltpu.VMEM((CH, HC), jnp.float32),
            pltpu.VMEM((CH,), jnp.int32),
            pltpu.VMEM((16, HC), jnp.float32),
            pltpu.VMEM_SHARED((NP, HC), jnp.float32),
            pltpu.VMEM_SHARED((DN_ROWS, HC), jnp.float32),
            pltpu.SemaphoreType.DMA,
            pltpu.SemaphoreType.DMA,
            pltpu.SemaphoreType.DMA,
        ],
    )
    return f(kv, q, e_full, src3, dst3)


def _tc3_body(p_ref, pd_ref, s_ref, o_ref):
    av = p_ref[0] + p_ref[1]
    dd = pd_ref[0] + pd_ref[1]
    d0 = dd[:, 0:1]
    d1 = dd[:, 1:2]
    o0 = jnp.where(d0 > 0, av[:, :C] / d0, 0.0)
    o1 = jnp.where(d1 > 0, av[:, C:] / d1, 0.0)
    o_ref[...] = jnp.concatenate([o0, o1], axis=1) + s_ref[...]


def _tc3(parts, dn2, skip):
    BLK = 1000
    grid = (N // BLK,)
    return pl.pallas_call(
        _tc3_body,
        grid=grid,
        in_specs=[pl.BlockSpec((2, BLK, HC), lambda i: (0, i, 0)),
                  pl.BlockSpec((2, BLK, 2), lambda i: (0, i, 0)),
                  pl.BlockSpec((BLK, HC), lambda i: (i, 0))],
        out_specs=pl.BlockSpec((BLK, HC), lambda i: (i, 0)),
        out_shape=jax.ShapeDtypeStruct((N, HC), jnp.float32),
    )(parts, dn2, skip)


def kernel(x, last_update, edge_index, t, msg, W_t, b_t, Wq, bq, Wk, bk, Wv, bv, We, Wskip, bskip):
    src1 = edge_index[0]
    dst1 = edge_index[1]
    src3 = src1.reshape(NW, NCH, CH)
    kv, q, skip = _tc1(x, Wk, bk, Wv, bv, Wq, bq, Wskip, bskip)
    lu_src = _sc0(last_update, src3).reshape(E)
    e_full = _tc2(lu_src, t, msg, W_t, b_t, We)
    parts, dn_parts = _sc1(kv, q, e_full, src1, dst1)
    dn2 = dn_parts.reshape(2, DN_ROWS * C, 2)
    return _tc3(parts, dn2, skip)


# speedup vs baseline: 19.8354x; 1.0659x over previous
import jax
import jax.numpy as jnp
import numpy as np
from jax import lax
from jax.experimental import pallas as pl
from jax.experimental.pallas import tpu as pltpu
from jax.experimental.pallas import tpu_sc as plsc

N = 10000
E = 320000
D = 128
MSG = 16
TIME = 16
H = 2
C = 64
HC = H * C

NW = 32
EPW = E // NW
CH = 80
NCH = EPW // CH
NP = 10240
TPS = NP // 16
DN_ROWS = 256


def _tc1_body(x_ref, wk_ref, wv_ref, wq_ref, ws_ref, bk_ref, bv_ref, bq_ref, bs_ref,
              kv_ref, q_ref, s_ref):
    x = x_ref[...]
    kv_ref[:, :HC] = jnp.dot(x, wk_ref[...], preferred_element_type=jnp.float32) + bk_ref[...]
    kv_ref[:, HC:] = jnp.dot(x, wv_ref[...], preferred_element_type=jnp.float32) + bv_ref[...]
    q_ref[...] = jnp.dot(x, wq_ref[...], preferred_element_type=jnp.float32) + bq_ref[...]
    s_ref[...] = jnp.dot(x, ws_ref[...], preferred_element_type=jnp.float32) + bs_ref[...]


def _tc1(x, Wk, bk, Wv, bv, Wq, bq, Wskip, bskip):
    BLK = 1000
    grid = (N // BLK,)
    w_spec = pl.BlockSpec((D, HC), lambda i: (0, 0))
    b_spec = pl.BlockSpec((HC,), lambda i: (0,))
    kv, q, s = pl.pallas_call(
        _tc1_body,
        grid=grid,
        in_specs=[pl.BlockSpec((BLK, D), lambda i: (i, 0)),
                  w_spec, w_spec, w_spec, w_spec,
                  b_spec, b_spec, b_spec, b_spec],
        out_specs=[pl.BlockSpec((BLK, 2 * HC), lambda i: (i, 0)),
                   pl.BlockSpec((BLK, HC), lambda i: (i, 0)),
                   pl.BlockSpec((BLK, HC), lambda i: (i, 0))],
        out_shape=[jax.ShapeDtypeStruct((N, 2 * HC), jnp.float32),
                   jax.ShapeDtypeStruct((N, HC), jnp.float32),
                   jax.ShapeDtypeStruct((N, HC), jnp.float32)],
    )(x, Wk, Wv, Wq, Wskip, bk, bv, bq, bskip)
    return kv, q, s


def _sc0_body(lu_hbm, src_hbm, out_hbm, idx_v, lu_v, sem):
    c = lax.axis_index("c")
    s = lax.axis_index("s")
    wid = c * 16 + s
    pltpu.sync_copy(src_hbm.at[wid], idx_v)

    def group(g, carry):
        cps = [pltpu.async_copy(lu_hbm.at[idx_v.at[5 * g + k]],
                                lu_v.at[5 * g + k], sem) for k in range(5)]
        for cp in cps:
            cp.wait()
        return carry

    lax.fori_loop(0, NCH // 5, group, 0)
    pltpu.sync_copy(lu_v, out_hbm.at[wid])


def _sc0(last_update, src3):
    mesh = plsc.VectorSubcoreMesh(core_axis_name="c", subcore_axis_name="s")
    f = pl.kernel(
        _sc0_body,
        mesh=mesh,
        out_type=jax.ShapeDtypeStruct((NW, NCH, CH), jnp.float32),
        scratch_types=[
            pltpu.VMEM((NCH, CH), jnp.int32),
            pltpu.VMEM((NCH, CH), jnp.float32),
            pltpu.SemaphoreType.DMA,
        ],
    )
    return f(last_update, src3)


def _tc2_body(lu_ref, t_ref, msg_ref, wt_ref, bt_ref, we_ref, e_ref):
    rel = lu_ref[...] - t_ref[...]
    enc = jnp.cos(rel[:, None] * wt_ref[0][None, :] + bt_ref[...][None, :])
    e_ref[...] = (jnp.dot(enc, we_ref[:TIME, :], preferred_element_type=jnp.float32)
                  + jnp.dot(msg_ref[...], we_ref[TIME:, :], preferred_element_type=jnp.float32))


def _tc2(lu_src, t, msg, W_t, b_t, We):
    BLK = 512
    grid = (E // BLK,)
    return pl.pallas_call(
        _tc2_body,
        grid=grid,
        in_specs=[pl.BlockSpec((BLK,), lambda i: (i,)),
                  pl.BlockSpec((BLK,), lambda i: (i,)),
                  pl.BlockSpec((BLK, MSG), lambda i: (i, 0)),
                  pl.BlockSpec((1, TIME), lambda i: (0, 0)),
                  pl.BlockSpec((TIME,), lambda i: (0,)),
                  pl.BlockSpec((TIME + MSG, HC), lambda i: (0, 0))],
        out_specs=pl.BlockSpec((BLK, HC), lambda i: (i, 0)),
        out_shape=jax.ShapeDtypeStruct((E, HC), jnp.float32),
    )(lu_src, t, msg, W_t, b_t, We)


def _sc1_body(kv_hbm, q_hbm, e_hbm, src_hbm, dst_hbm, parts_hbm, dn_hbm,
              sidx_c, didx_c, kv_v, q_v, e_v, d64_v, zrow_v,
              acc_sh, dn_sh, sem_kv, sem_q, sem_e, sem_si, sem_di, sem_av, sem_dn):
    c = lax.axis_index("c")
    s = lax.axis_index("s")
    wid = c * 16 + s

    def zlane(i, carry):
        for j in range(HC // 16):
            zrow_v[i, pl.ds(j * 16, 16)] = jnp.zeros((16,), jnp.float32)
        return carry

    lax.fori_loop(0, 16, zlane, 0)

    def zacc(r, carry):
        pltpu.sync_copy(zrow_v, acc_sh.at[pl.ds(s * TPS + r * 16, 16)])
        return carry

    lax.fori_loop(0, TPS // 16, zacc, 0)
    pltpu.sync_copy(zrow_v, dn_sh.at[pl.ds(s * 16, 16)])
    plsc.subcore_barrier()

    inv_sqrt_c = np.float32(1.0 / np.sqrt(C))
    lane = lax.iota(jnp.int32, 16)

    def lane_sum(vec):
        for sft in (1, 2, 4, 8):
            vec = vec + vec.at[jnp.bitwise_xor(lane, sft)].get(mode="promise_in_bounds")
        return vec

    def chunk(j, carry):
        ebase = pl.multiple_of(wid * EPW + j * CH, 8)
        cp_si = pltpu.async_copy(src_hbm.at[pl.ds(ebase, CH)], sidx_c, sem_si)
        cp_di = pltpu.async_copy(dst_hbm.at[pl.ds(ebase, CH)], didx_c, sem_di)
        cp_e = pltpu.async_copy(e_hbm.at[pl.ds(ebase, CH)], e_v, sem_e)
        cp_si.wait()
        cp_di.wait()
        cp_kv = pltpu.async_copy(kv_hbm.at[sidx_c], kv_v, sem_kv)
        cp_q = pltpu.async_copy(q_hbm.at[didx_c], q_v, sem_q)
        cp_kv.wait()
        cp_q.wait()
        cp_e.wait()

        for g in range(CH // 16):
            d16 = didx_c[pl.ds(g * 16, 16)]
            d64_v[pl.ds(g * 16, 16)] = lax.shift_right_logical(d16, 6)

        def edge(i, carry2):
            acc0 = jnp.zeros((16,), jnp.float32)
            acc1 = jnp.zeros((16,), jnp.float32)
            for jj in range(4):
                qv = q_v[i, pl.ds(jj * 16, 16)]
                kv = kv_v[i, pl.ds(jj * 16, 16)] + e_v[i, pl.ds(jj * 16, 16)]
                acc0 = acc0 + qv * kv
            for jj in range(4, 8):
                qv = q_v[i, pl.ds(jj * 16, 16)]
                kv = kv_v[i, pl.ds(jj * 16, 16)] + e_v[i, pl.ds(jj * 16, 16)]
                acc1 = acc1 + qv * kv
            a0v = jnp.exp(lane_sum(acc0) * inv_sqrt_c)
            a1v = jnp.exp(lane_sum(acc1) * inv_sqrt_c)
            for jj in range(4):
                e_v[i, pl.ds(jj * 16, 16)] = a0v * (
                    kv_v[i, pl.ds(HC + jj * 16, 16)] + e_v[i, pl.ds(jj * 16, 16)])
            for jj in range(4, 8):
                e_v[i, pl.ds(jj * 16, 16)] = a1v * (
                    kv_v[i, pl.ds(HC + jj * 16, 16)] + e_v[i, pl.ds(jj * 16, 16)])
            dgrp = didx_c[pl.ds((i // 16) * 16, 16)]
            dstv = dgrp.at[jnp.full((16,), i % 16, jnp.int32)].get(
                mode="promise_in_bounds")
            col0 = 2 * jnp.bitwise_and(dstv, 63)
            zero16 = jnp.zeros((16,), jnp.float32)
            for jj in range(8):
                lane16 = lane + (16 * jj)
                q_v[i, pl.ds(jj * 16, 16)] = jnp.where(
                    lane16 == col0, a0v,
                    jnp.where(lane16 == col0 + 1, a1v, zero16))
            return carry2

        lax.fori_loop(0, CH, edge, 0)
        cp_av = pltpu.async_copy(e_v, acc_sh.at[didx_c], sem_av, add=True)
        cp_dn = pltpu.async_copy(q_v, dn_sh.at[d64_v], sem_dn, add=True)
        cp_av.wait()
        cp_dn.wait()
        return carry

    lax.fori_loop(0, NCH, chunk, 0)
    plsc.subcore_barrier()

    pltpu.sync_copy(acc_sh.at[pl.ds(s * TPS, TPS)],
                    parts_hbm.at[c, pl.ds(s * TPS, TPS)])
    pltpu.sync_copy(dn_sh.at[pl.ds(s * 16, 16)],
                    dn_hbm.at[c, pl.ds(s * 16, 16)])


def _sc1(kv, q, e_full, src1, dst1):
    mesh = plsc.VectorSubcoreMesh(core_axis_name="c", subcore_axis_name="s")
    f = pl.kernel(
        _sc1_body,
        mesh=mesh,
        out_type=[jax.ShapeDtypeStruct((2, NP, HC), jnp.float32),
                  jax.ShapeDtypeStruct((2, DN_ROWS, HC), jnp.float32)],
        scratch_types=[
            pltpu.VMEM((CH,), jnp.int32),
            pltpu.VMEM((CH,), jnp.int32),
            pltpu.VMEM((CH, 2 * HC), jnp.float32),
            pltpu.VMEM((CH, HC), jnp.float32),
            pltpu.VMEM((CH, HC), jnp.float32),
            pltpu.VMEM((CH,), jnp.int32),
            pltpu.VMEM((16, HC), jnp.float32),
            pltpu.VMEM_SHARED((NP, HC), jnp.float32),
            pltpu.VMEM_SHARED((DN_ROWS, HC), jnp.float32),
            pltpu.SemaphoreType.DMA, pltpu.SemaphoreType.DMA,
            pltpu.SemaphoreType.DMA, pltpu.SemaphoreType.DMA,
            pltpu.SemaphoreType.DMA, pltpu.SemaphoreType.DMA,
            pltpu.SemaphoreType.DMA,
        ],
    )
    return f(kv, q, e_full, src1, dst1)


def _tc3_body(p_ref, pd_ref, s_ref, o_ref):
    av = p_ref[0] + p_ref[1]
    dd = pd_ref[0] + pd_ref[1]
    d0 = dd[:, 0:1]
    d1 = dd[:, 1:2]
    o0 = jnp.where(d0 > 0, av[:, :C] / d0, 0.0)
    o1 = jnp.where(d1 > 0, av[:, C:] / d1, 0.0)
    o_ref[...] = jnp.concatenate([o0, o1], axis=1) + s_ref[...]


def _tc3(parts, dn2, skip):
    BLK = 1000
    grid = (N // BLK,)
    return pl.pallas_call(
        _tc3_body,
        grid=grid,
        in_specs=[pl.BlockSpec((2, BLK, HC), lambda i: (0, i, 0)),
                  pl.BlockSpec((2, BLK, 2), lambda i: (0, i, 0)),
                  pl.BlockSpec((BLK, HC), lambda i: (i, 0))],
        out_specs=pl.BlockSpec((BLK, HC), lambda i: (i, 0)),
        out_shape=jax.ShapeDtypeStruct((N, HC), jnp.float32),
    )(parts, dn2, skip)


def kernel(x, last_update, edge_index, t, msg, W_t, b_t, Wq, bq, Wk, bk, Wv, bv, We, Wskip, bskip):
    src1 = edge_index[0]
    dst1 = edge_index[1]
    src3 = src1.reshape(NW, NCH, CH)
    kv, q, skip = _tc1(x, Wk, bk, Wv, bv, Wq, bq, Wskip, bskip)
    lu_src = _sc0(last_update, src3).reshape(E)
    e_full = _tc2(lu_src, t, msg, W_t, b_t, We)
    parts, dn_parts = _sc1(kv, q, e_full, src1, dst1)
    dn2 = dn_parts.reshape(2, DN_ROWS * C, 2)
    return _tc3(parts, dn2, skip)


# speedup vs baseline: 20.9464x; 1.0560x over previous
import jax
import jax.numpy as jnp
import numpy as np
from jax import lax
from jax.experimental import pallas as pl
from jax.experimental.pallas import tpu as pltpu
from jax.experimental.pallas import tpu_sc as plsc

N = 10000
E = 320000
D = 128
MSG = 16
TIME = 16
H = 2
C = 64
HC = H * C

NW = 32
EPW = E // NW
CH = 80
NCH = EPW // CH
NP = 10240
TPS = NP // 16
DN_ROWS = 256


def _tc1_body(x_ref, wk_ref, wv_ref, wq_ref, ws_ref, bk_ref, bv_ref, bq_ref, bs_ref,
              kv_ref, q_ref, s_ref):
    x = x_ref[...]
    kv_ref[:, :HC] = jnp.dot(x, wk_ref[...], preferred_element_type=jnp.float32) + bk_ref[...]
    kv_ref[:, HC:] = jnp.dot(x, wv_ref[...], preferred_element_type=jnp.float32) + bv_ref[...]
    q_ref[...] = jnp.dot(x, wq_ref[...], preferred_element_type=jnp.float32) + bq_ref[...]
    s_ref[...] = jnp.dot(x, ws_ref[...], preferred_element_type=jnp.float32) + bs_ref[...]


def _tc1(x, Wk, bk, Wv, bv, Wq, bq, Wskip, bskip):
    BLK = 1000
    grid = (N // BLK,)
    w_spec = pl.BlockSpec((D, HC), lambda i: (0, 0))
    b_spec = pl.BlockSpec((HC,), lambda i: (0,))
    kv, q, s = pl.pallas_call(
        _tc1_body,
        grid=grid,
        in_specs=[pl.BlockSpec((BLK, D), lambda i: (i, 0)),
                  w_spec, w_spec, w_spec, w_spec,
                  b_spec, b_spec, b_spec, b_spec],
        out_specs=[pl.BlockSpec((BLK, 2 * HC), lambda i: (i, 0)),
                   pl.BlockSpec((BLK, HC), lambda i: (i, 0)),
                   pl.BlockSpec((BLK, HC), lambda i: (i, 0))],
        out_shape=[jax.ShapeDtypeStruct((N, 2 * HC), jnp.float32),
                   jax.ShapeDtypeStruct((N, HC), jnp.float32),
                   jax.ShapeDtypeStruct((N, HC), jnp.float32)],
    )(x, Wk, Wv, Wq, Wskip, bk, bv, bq, bskip)
    return kv, q, s


def _sc0_body(lu_hbm, src_hbm, out_hbm, idx_v, lu_v, sem):
    c = lax.axis_index("c")
    s = lax.axis_index("s")
    wid = c * 16 + s
    pltpu.sync_copy(src_hbm.at[wid], idx_v)

    def group(g, carry):
        cps = [pltpu.async_copy(lu_hbm.at[idx_v.at[5 * g + k]],
                                lu_v.at[5 * g + k], sem) for k in range(5)]
        for cp in cps:
            cp.wait()
        return carry

    lax.fori_loop(0, NCH // 5, group, 0)
    pltpu.sync_copy(lu_v, out_hbm.at[wid])


def _sc0(last_update, src3):
    mesh = plsc.VectorSubcoreMesh(core_axis_name="c", subcore_axis_name="s")
    f = pl.kernel(
        _sc0_body,
        mesh=mesh,
        out_type=jax.ShapeDtypeStruct((NW, NCH, CH), jnp.float32),
        scratch_types=[
            pltpu.VMEM((NCH, CH), jnp.int32),
            pltpu.VMEM((NCH, CH), jnp.float32),
            pltpu.SemaphoreType.DMA,
        ],
    )
    return f(last_update, src3)


def _tc2_body(lu_ref, t_ref, msg_ref, wt_ref, bt_ref, we_ref, e_ref):
    rel = lu_ref[0, 0] - t_ref[0, 0]
    enc = jnp.cos(rel[:, None] * wt_ref[0][None, :] + bt_ref[...][None, :])
    e_ref[...] = (jnp.dot(enc, we_ref[:TIME, :], preferred_element_type=jnp.float32)
                  + jnp.dot(msg_ref[...], we_ref[TIME:, :], preferred_element_type=jnp.float32))


def _tc2(lu_src, t, msg, W_t, b_t, We):
    BLK = 2560
    grid = (E // BLK,)
    lu2 = lu_src.reshape(E // BLK, 1, BLK)
    t2 = t.reshape(E // BLK, 1, BLK)
    return pl.pallas_call(
        _tc2_body,
        grid=grid,
        in_specs=[pl.BlockSpec((1, 1, BLK), lambda i: (i, 0, 0)),
                  pl.BlockSpec((1, 1, BLK), lambda i: (i, 0, 0)),
                  pl.BlockSpec((BLK, MSG), lambda i: (i, 0)),
                  pl.BlockSpec((1, TIME), lambda i: (0, 0)),
                  pl.BlockSpec((TIME,), lambda i: (0,)),
                  pl.BlockSpec((TIME + MSG, HC), lambda i: (0, 0))],
        out_specs=pl.BlockSpec((BLK, HC), lambda i: (i, 0)),
        out_shape=jax.ShapeDtypeStruct((E, HC), jnp.float32),
    )(lu2, t2, msg, W_t, b_t, We)


def _sc1_body(kv_hbm, q_hbm, e_hbm, src_hbm, dst_hbm, parts_hbm, dn_hbm,
              sidx_c, didx_c, kv_v, q_v, e_v, d64_v, zrow_v,
              acc_sh, dn_sh, sem_kv, sem_q, sem_e, sem_si, sem_di, sem_av, sem_dn):
    c = lax.axis_index("c")
    s = lax.axis_index("s")
    wid = c * 16 + s

    def zlane(i, carry):
        for j in range(HC // 16):
            zrow_v[i, pl.ds(j * 16, 16)] = jnp.zeros((16,), jnp.float32)
        return carry

    lax.fori_loop(0, 16, zlane, 0)

    def zacc(r, carry):
        pltpu.sync_copy(zrow_v, acc_sh.at[pl.ds(s * TPS + r * 16, 16)])
        return carry

    lax.fori_loop(0, TPS // 16, zacc, 0)
    pltpu.sync_copy(zrow_v, dn_sh.at[pl.ds(s * 16, 16)])
    plsc.subcore_barrier()

    inv_sqrt_c = np.float32(1.0 / np.sqrt(C))
    lane = lax.iota(jnp.int32, 16)

    def lane_sum(vec):
        for sft in (1, 2, 4, 8):
            vec = vec + vec.at[jnp.bitwise_xor(lane, sft)].get(mode="promise_in_bounds")
        return vec

    def chunk(j, carry):
        ebase = pl.multiple_of(wid * EPW + j * CH, 8)
        cp_si = pltpu.async_copy(src_hbm.at[pl.ds(ebase, CH)], sidx_c, sem_si)
        cp_di = pltpu.async_copy(dst_hbm.at[pl.ds(ebase, CH)], didx_c, sem_di)
        cp_e = pltpu.async_copy(e_hbm.at[pl.ds(ebase, CH)], e_v, sem_e)
        cp_si.wait()
        cp_di.wait()
        cp_kv = pltpu.async_copy(kv_hbm.at[sidx_c], kv_v, sem_kv)
        cp_q = pltpu.async_copy(q_hbm.at[didx_c], q_v, sem_q)
        cp_kv.wait()
        cp_q.wait()
        cp_e.wait()

        for g in range(CH // 16):
            d16 = didx_c[pl.ds(g * 16, 16)]
            d64_v[pl.ds(g * 16, 16)] = lax.shift_right_logical(d16, 6)

        def edge(i, carry2):
            acc0 = jnp.zeros((16,), jnp.float32)
            acc1 = jnp.zeros((16,), jnp.float32)
            for jj in range(4):
                qv = q_v[i, pl.ds(jj * 16, 16)]
                kv = kv_v[i, pl.ds(jj * 16, 16)] + e_v[i, pl.ds(jj * 16, 16)]
                acc0 = acc0 + qv * kv
            for jj in range(4, 8):
                qv = q_v[i, pl.ds(jj * 16, 16)]
                kv = kv_v[i, pl.ds(jj * 16, 16)] + e_v[i, pl.ds(jj * 16, 16)]
                acc1 = acc1 + qv * kv
            a0v = jnp.exp(lane_sum(acc0) * inv_sqrt_c)
            a1v = jnp.exp(lane_sum(acc1) * inv_sqrt_c)
            for jj in range(4):
                e_v[i, pl.ds(jj * 16, 16)] = a0v * (
                    kv_v[i, pl.ds(HC + jj * 16, 16)] + e_v[i, pl.ds(jj * 16, 16)])
            for jj in range(4, 8):
                e_v[i, pl.ds(jj * 16, 16)] = a1v * (
                    kv_v[i, pl.ds(HC + jj * 16, 16)] + e_v[i, pl.ds(jj * 16, 16)])
            dgrp = didx_c[pl.ds((i // 16) * 16, 16)]
            dstv = dgrp.at[jnp.full((16,), i % 16, jnp.int32)].get(
                mode="promise_in_bounds")
            col0 = 2 * jnp.bitwise_and(dstv, 63)
            zero16 = jnp.zeros((16,), jnp.float32)
            for jj in range(8):
                lane16 = lane + (16 * jj)
                q_v[i, pl.ds(jj * 16, 16)] = jnp.where(
                    lane16 == col0, a0v,
                    jnp.where(lane16 == col0 + 1, a1v, zero16))
            return carry2

        lax.fori_loop(0, CH, edge, 0)
        cp_av = pltpu.async_copy(e_v, acc_sh.at[didx_c], sem_av, add=True)
        cp_dn = pltpu.async_copy(q_v, dn_sh.at[d64_v], sem_dn, add=True)
        cp_av.wait()
        cp_dn.wait()
        return carry

    lax.fori_loop(0, NCH, chunk, 0)
    plsc.subcore_barrier()

    pltpu.sync_copy(acc_sh.at[pl.ds(s * TPS, TPS)],
                    parts_hbm.at[c, pl.ds(s * TPS, TPS)])
    pltpu.sync_copy(dn_sh.at[pl.ds(s * 16, 16)],
                    dn_hbm.at[c, pl.ds(s * 16, 16)])


def _sc1(kv, q, e_full, src1, dst1):
    mesh = plsc.VectorSubcoreMesh(core_axis_name="c", subcore_axis_name="s")
    f = pl.kernel(
        _sc1_body,
        mesh=mesh,
        out_type=[jax.ShapeDtypeStruct((2, NP, HC), jnp.float32),
                  jax.ShapeDtypeStruct((2, DN_ROWS, HC), jnp.float32)],
        scratch_types=[
            pltpu.VMEM((CH,), jnp.int32),
            pltpu.VMEM((CH,), jnp.int32),
            pltpu.VMEM((CH, 2 * HC), jnp.float32),
            pltpu.VMEM((CH, HC), jnp.float32),
            pltpu.VMEM((CH, HC), jnp.float32),
            pltpu.VMEM((CH,), jnp.int32),
            pltpu.VMEM((16, HC), jnp.float32),
            pltpu.VMEM_SHARED((NP, HC), jnp.float32),
            pltpu.VMEM_SHARED((DN_ROWS, HC), jnp.float32),
            pltpu.SemaphoreType.DMA, pltpu.SemaphoreType.DMA,
            pltpu.SemaphoreType.DMA, pltpu.SemaphoreType.DMA,
            pltpu.SemaphoreType.DMA, pltpu.SemaphoreType.DMA,
            pltpu.SemaphoreType.DMA,
        ],
    )
    return f(kv, q, e_full, src1, dst1)


def _tc3_body(p_ref, pd_ref, s_ref, o_ref):
    av = p_ref[0] + p_ref[1]
    dd = pd_ref[0] + pd_ref[1]
    d0 = dd[:, 0:1]
    d1 = dd[:, 1:2]
    o0 = jnp.where(d0 > 0, av[:, :C] / d0, 0.0)
    o1 = jnp.where(d1 > 0, av[:, C:] / d1, 0.0)
    o_ref[...] = jnp.concatenate([o0, o1], axis=1) + s_ref[...]


def _tc3(parts, dn2, skip):
    BLK = 1000
    grid = (N // BLK,)
    return pl.pallas_call(
        _tc3_body,
        grid=grid,
        in_specs=[pl.BlockSpec((2, BLK, HC), lambda i: (0, i, 0)),
                  pl.BlockSpec((2, BLK, 2), lambda i: (0, i, 0)),
                  pl.BlockSpec((BLK, HC), lambda i: (i, 0))],
        out_specs=pl.BlockSpec((BLK, HC), lambda i: (i, 0)),
        out_shape=jax.ShapeDtypeStruct((N, HC), jnp.float32),
    )(parts, dn2, skip)


def kernel(x, last_update, edge_index, t, msg, W_t, b_t, Wq, bq, Wk, bk, Wv, bv, We, Wskip, bskip):
    src1 = edge_index[0]
    dst1 = edge_index[1]
    src3 = src1.reshape(NW, NCH, CH)
    kv, q, skip = _tc1(x, Wk, bk, Wv, bv, Wq, bq, Wskip, bskip)
    lu_src = _sc0(last_update, src3).reshape(E)
    e_full = _tc2(lu_src, t, msg, W_t, b_t, We)
    parts, dn_parts = _sc1(kv, q, e_full, src1, dst1)
    dn2 = dn_parts.reshape(2, DN_ROWS * C, 2)
    return _tc3(parts, dn2, skip)
